# baseline (speedup 1.0000x reference)
import functools

import jax
import jax.numpy as jnp
from jax import lax
from jax.experimental import pallas as pl
from jax.experimental.pallas import tpu as pltpu
from jax.experimental.pallas import tpu_sc as plsc

_L = 16


def _make_gather(d, nw, ch, cw, lpb, t):
    per_w = ch * cw
    mesh = plsc.VectorSubcoreMesh(core_axis_name="c", subcore_axis_name="s")
    nc = mesh.num_cores

    mul, sh = 5243, 18
    assert lpb == 50 and per_w <= 6400

    @functools.partial(
        pl.kernel,
        out_type=jax.ShapeDtypeStruct((nw * per_w, d), jnp.float32),
        mesh=mesh,
        scratch_types=[
            pltpu.VMEM((ch, cw), jnp.int32),
            pltpu.VMEM((2, cw, d), jnp.float32),
            pltpu.SemaphoreType.DMA,
            pltpu.SemaphoreType.DMA,
        ],
        compiler_params=pltpu.CompilerParams(use_tc_tiling_on_sc=False),
    )
    def k(bounds_hbm, table_hbm, out_hbm, idx_v, rows_v, gsem, osem):
        wid = lax.axis_index("s") * nc + lax.axis_index("c")
        pltpu.sync_copy(bounds_hbm.at[wid], idx_v)
        wbase = wid * (per_w // lpb)

        lane = lax.iota(jnp.int32, 16)

        def compute_idx(j, carry):
            for kk in range(cw // _L):
                n = j * cw + kk * _L + lane
                b = lax.shift_right_logical(n * mul, sh)
                raw = idx_v[j, pl.ds(kk * _L, _L)]
                g = lax.shift_right_logical(raw, 2) + (wbase + b) * t
                idx_v[j, pl.ds(kk * _L, _L)] = g
            return carry

        lax.fori_loop(0, ch, compute_idx, 0)

        def gather(j, buf):
            return pltpu.async_copy(
                table_hbm.at[idx_v.at[j]], rows_v.at[buf], gsem)

        gather(0, 0)

        def step(j, carry):
            buf = lax.rem(j, 2)
            pltpu.make_async_copy(
                table_hbm.at[idx_v.at[j]], rows_v.at[buf], gsem).wait()

            @pl.when(j < ch - 1)
            def _():
                gather(j + 1, 1 - buf)

            pltpu.async_copy(
                rows_v.at[buf],
                out_hbm.at[pl.ds(wid * per_w + j * cw, cw)],
                osem).wait()
            return carry

        lax.fori_loop(0, ch, step, 0)

    return k


def _make_tc_gather(T, D, L, bb):
    def tck(b_ref, x_ref, o_ref):
        rows = lax.shift_right_logical(b_ref[...], 2)
        oh = (rows[:, :, None]
              == lax.broadcasted_iota(jnp.int32, (bb, L, T), 2)
              ).astype(jnp.float32)
        o_ref[...] = lax.dot_general(
            oh, x_ref[...],
            dimension_numbers=(((2,), (1,)), ((0,), (0,))),
            precision=lax.Precision.HIGHEST,
            preferred_element_type=jnp.float32)

    def fn(bounds, X):
        Bt = X.shape[0]
        return pl.pallas_call(
            tck,
            grid=(Bt // bb,),
            in_specs=[
                pl.BlockSpec((bb, L), lambda i: (i, 0)),
                pl.BlockSpec((bb, T, D), lambda i: (i, 0, 0)),
            ],
            out_specs=pl.BlockSpec((bb, L, D), lambda i: (i, 0, 0)),
            out_shape=jax.ShapeDtypeStruct((Bt, L, D), jnp.float32),
        )(bounds, X)

    return fn


def kernel(X, bounds):
    B, T, D = X.shape
    Bb, L = bounds.shape
    assert B == Bb
    bounds = bounds.astype(jnp.int32)

    B_sc = B // 2
    NW = 32
    per_w = B_sc * L // NW
    CW = 128
    CH = per_w // CW
    table = X[:B_sc].reshape(B_sc * T, D)
    b3 = bounds[:B_sc].reshape(NW, CH, CW)
    sc_out = _make_gather(D, NW, CH, CW, L, T)(b3, table)
    tc_out = _make_tc_gather(T, D, L, 32)(bounds[B_sc:], X[B_sc:])
    return jnp.concatenate([sc_out.reshape(B_sc, L, D), tc_out], axis=0)


# speedup vs baseline: 1.4816x; 1.4816x over previous
import functools

import jax
import jax.numpy as jnp
from jax import lax
from jax.experimental import pallas as pl
from jax.experimental.pallas import tpu as pltpu
from jax.experimental.pallas import tpu_sc as plsc

_L = 16


def _make_gather(d, nw, ch, cw, lpb, t):
    per_w = ch * cw
    mesh = plsc.VectorSubcoreMesh(core_axis_name="c", subcore_axis_name="s")
    nc = mesh.num_cores

    mul, sh = 5243, 18
    assert lpb == 50 and per_w <= 6400

    @functools.partial(
        pl.kernel,
        out_type=jax.ShapeDtypeStruct((nw * per_w, d), jnp.float32),
        mesh=mesh,
        scratch_types=[
            pltpu.VMEM((ch, cw), jnp.int32),
            pltpu.VMEM((2, cw, d), jnp.float32),
            pltpu.SemaphoreType.DMA,
            pltpu.SemaphoreType.DMA,
        ],
        compiler_params=pltpu.CompilerParams(use_tc_tiling_on_sc=False),
    )
    def k(bounds_hbm, table_hbm, out_hbm, idx_v, rows_v, gsem, osem):
        wid = lax.axis_index("s") * nc + lax.axis_index("c")
        pltpu.sync_copy(bounds_hbm.at[wid], idx_v)
        wbase = wid * (per_w // lpb)

        lane = lax.iota(jnp.int32, 16)

        def compute_idx(j, carry):
            for kk in range(cw // _L):
                n = j * cw + kk * _L + lane
                b = lax.shift_right_logical(n * mul, sh)
                raw = idx_v[j, pl.ds(kk * _L, _L)]
                g = lax.shift_right_logical(raw, 2) + (wbase + b) * t
                idx_v[j, pl.ds(kk * _L, _L)] = g
            return carry

        lax.fori_loop(0, ch, compute_idx, 0)

        def gather(j, buf):
            return pltpu.async_copy(
                table_hbm.at[idx_v.at[j]], rows_v.at[buf], gsem)

        gather(0, 0)

        def step(j, carry):
            buf = lax.rem(j, 2)
            pltpu.make_async_copy(
                table_hbm.at[idx_v.at[j]], rows_v.at[buf], gsem).wait()

            @pl.when(j < ch - 1)
            def _():
                gather(j + 1, 1 - buf)

            pltpu.async_copy(
                rows_v.at[buf],
                out_hbm.at[pl.ds(wid * per_w + j * cw, cw)],
                osem).wait()
            return carry

        lax.fori_loop(0, ch, step, 0)

    return k


def kernel(X, bounds):
    B, T, D = X.shape
    Bb, L = bounds.shape
    NW = 32
    assert B == Bb and (B * L) % NW == 0
    per_w = B * L // NW
    CW = 128
    CH = per_w // CW
    table = X.reshape(B * T, D)
    b3 = bounds.astype(jnp.int32).reshape(NW, CH, CW)
    fn = _make_gather(D, NW, CH, CW, L, T)
    out = fn(b3, table)
    return out.reshape(B, L, D)
